# baseline (speedup 1.0000x reference)
import functools

import jax
import jax.numpy as jnp
from jax import lax
from jax.experimental import pallas as pl
from jax.experimental.pallas import tpu as pltpu
from jax.experimental.pallas import tpu_sc as plsc

N_NODES = 10000
D = 128
DH = D // 2
NUM_GRAPHS = 64
NC = 2
NS = 16
NW = NC * NS
CH = 128

KD = 80
EPAD_D = NW * KD * CH

KP = 162
EPAD_P = NS * KP * CH

ZROWS = 10240
RPT = ZROWS // NS

_mesh = plsc.VectorSubcoreMesh(core_axis_name="c", subcore_axis_name="s")


@functools.partial(
    pl.kernel,
    out_type=jax.ShapeDtypeStruct((NC, N_NODES), jnp.float32),
    mesh=_mesh,
    scratch_types=[
        pltpu.VMEM((KD, CH), jnp.int32),
        pltpu.VMEM((KD, CH), jnp.float32),
        pltpu.VMEM((N_NODES,), jnp.float32),
        pltpu.VMEM_SHARED((N_NODES,), jnp.float32),
    ],
)
def _sc_deg(dst_hbm, w_hbm, deg_hbm, dst_v, w_v, zbuf, deg_sh):
    cid = lax.axis_index("c")
    sid = lax.axis_index("s")
    wid = cid * NS + sid

    @pl.when(sid == 0)
    def _():
        @pl.loop(0, N_NODES // 16)
        def _(i):
            zbuf[pl.ds(i * 16, 16)] = jnp.zeros((16,), jnp.float32)

        pltpu.sync_copy(zbuf, deg_sh)

    plsc.subcore_barrier()

    pltpu.sync_copy(dst_hbm.at[wid], dst_v)
    pltpu.sync_copy(w_hbm.at[wid], w_v)

    @pl.loop(0, KD)
    def _(j):
        pltpu.sync_copy(w_v.at[j], deg_sh.at[dst_v.at[j]], add=True)

    plsc.subcore_barrier()

    @pl.when(sid == 0)
    def _():
        pltpu.sync_copy(deg_sh, deg_hbm.at[cid])


def _tc_y_body(x_ref, w_ref, degp_ref, y_ref):
    deg = degp_ref[:, 0:1] + degp_ref[:, 1:2] + 1.0
    dis = jnp.where(deg > 0, lax.rsqrt(deg), 0.0)
    xw = jnp.dot(x_ref[...], w_ref[...],
                 preferred_element_type=jnp.float32,
                 precision=lax.Precision.HIGHEST)
    ys = xw * dis
    y_ref[0:N_NODES, :] = ys[:, 0:DH]
    y_ref[N_NODES:2 * N_NODES, :] = ys[:, DH:D]


_tc_y = pl.pallas_call(
    _tc_y_body,
    out_shape=jax.ShapeDtypeStruct((2 * N_NODES, DH), jnp.float32),
)


@functools.partial(
    pl.kernel,
    out_type=jax.ShapeDtypeStruct((NC, ZROWS, DH), jnp.float32),
    mesh=_mesh,
    compiler_params=pltpu.CompilerParams(use_tc_tiling_on_sc=False),
    scratch_types=[
        pltpu.VMEM((KP, CH), jnp.int32),
        pltpu.VMEM((KP, CH), jnp.int32),
        pltpu.VMEM((KP, CH), jnp.float32),
        pltpu.VMEM((CH, DH), jnp.float32),
        pltpu.VMEM((CH, DH), jnp.float32),
        pltpu.VMEM((CH, DH), jnp.float32),
        pltpu.VMEM_SHARED((ZROWS, DH), jnp.float32),
        pltpu.SemaphoreType.DMA,
        pltpu.SemaphoreType.DMA,
        pltpu.SemaphoreType.DMA,
        pltpu.SemaphoreType.DMA,
        pltpu.SemaphoreType.DMA,
        pltpu.SemaphoreType.DMA,
    ],
)
def _sc_propagate(src_hbm, dst_hbm, w_hbm, y_hbm, z_hbm,
                  src_v, dst_v, w_v, b0, b1, b2, z_sh,
                  sg0, sg1, sg2, ss0, ss1, ss2):
    cid = lax.axis_index("c")
    sid = lax.axis_index("s")

    pltpu.async_copy(src_hbm.at[cid, sid], src_v, sg0)
    pltpu.async_copy(dst_hbm.at[sid], dst_v, sg1)
    pltpu.async_copy(w_hbm.at[sid], w_v, sg2)

    @pl.loop(0, CH)
    def _(r):
        for c in range(DH // 16):
            b0[r, pl.ds(c * 16, 16)] = jnp.zeros((16,), jnp.float32)

    for t in range(RPT // CH):
        pltpu.sync_copy(b0, z_sh.at[pl.ds(sid * RPT + t * CH, CH)])

    plsc.subcore_barrier()

    pltpu.make_async_copy(src_hbm.at[cid, sid], src_v, sg0).wait()
    pltpu.make_async_copy(dst_hbm.at[sid], dst_v, sg1).wait()
    pltpu.make_async_copy(w_hbm.at[sid], w_v, sg2).wait()

    bufs = (b0, b1, b2)
    gsems = (sg0, sg1, sg2)
    ssems = (ss0, ss1, ss2)

    def scale(j, bv):
        @pl.loop(0, CH // 16)
        def _(g):
            w16 = w_v[j, pl.ds(g * 16, 16)]
            for i in range(16):
                wr = w16[i]
                r = g * 16 + i
                for c in range(DH // 16):
                    sl = pl.ds(c * 16, 16)
                    bv[r, sl] = bv[r, sl] * wr

    def bank(j, b, guard_next, drain_prev):
        bv, gs, ss = bufs[b], gsems[b], ssems[b]
        nb = (b + 2) % 3
        pltpu.make_async_copy(y_hbm.at[src_v.at[j]], bv, gs).wait()
        scale(j, bv)
        pltpu.async_copy(bv, z_sh.at[dst_v.at[j]], ss, add=True)
        nbv, ngs, nss = bufs[nb], gsems[nb], ssems[nb]

        def issue_next():
            if drain_prev:
                pltpu.make_async_copy(
                    nbv, z_sh.at[dst_v.at[j]], nss).wait()
            pltpu.async_copy(y_hbm.at[src_v.at[j + 2]], nbv, ngs)

        if guard_next:
            pl.when(j + 2 < KP)(issue_next)
        else:
            issue_next()

    pltpu.async_copy(y_hbm.at[src_v.at[0]], b0, sg0)
    pltpu.async_copy(y_hbm.at[src_v.at[1]], b1, sg1)
    bank(0, 0, False, False)
    bank(1, 1, False, True)
    bank(2, 2, False, True)

    @pl.loop(3, KP, step=3)
    def _(t):
        bank(t, 0, False, True)
        bank(t + 1, 1, True, True)
        bank(t + 2, 2, True, True)

    for b in range(3):
        pltpu.make_async_copy(bufs[b], z_sh.at[dst_v.at[0]], ssems[b]).wait()

    plsc.subcore_barrier()
    pltpu.sync_copy(z_sh.at[pl.ds(sid * RPT, RPT)],
                    z_hbm.at[cid, pl.ds(sid * RPT, RPT)])


def _tc_final_body(z_ref, y_ref, degp_ref, b_ref, batch_ref, h_ref, g_ref):
    deg = degp_ref[:, 0:1] + degp_ref[:, 1:2] + 1.0
    dis = jnp.where(deg > 0, lax.rsqrt(deg), 0.0)
    z = jnp.concatenate(
        [z_ref[0, :N_NODES, :], z_ref[1, :N_NODES, :]], axis=1)
    y = jnp.concatenate(
        [y_ref[0:N_NODES, :], y_ref[N_NODES:2 * N_NODES, :]], axis=1)
    h = jnp.maximum((z + y) * dis + b_ref[...], 0.0)
    h_ref[...] = h
    iot = lax.broadcasted_iota(jnp.int32, (NUM_GRAPHS, N_NODES), 0)
    onehot = (batch_ref[...] == iot).astype(jnp.float32)
    counts = jnp.sum(onehot, axis=1, keepdims=True)
    sums = jnp.dot(onehot, h, preferred_element_type=jnp.float32,
                   precision=lax.Precision.HIGHEST)
    g_ref[...] = sums / jnp.maximum(counts, 1.0)


_tc_final = pl.pallas_call(
    _tc_final_body,
    out_shape=[
        jax.ShapeDtypeStruct((N_NODES, D), jnp.float32),
        jax.ShapeDtypeStruct((NUM_GRAPHS, D), jnp.float32),
    ],
)


def kernel(x, edge_index, edge_weight, batch, W, b):
    x = x.astype(jnp.float32)
    src = edge_index[0].astype(jnp.int32)
    dst = edge_index[1].astype(jnp.int32)
    w = edge_weight.astype(jnp.float32)
    e = src.shape[0]

    dst_d = jnp.pad(dst, (0, EPAD_D - e)).reshape(NW, KD, CH)
    w_d = jnp.pad(w, (0, EPAD_D - e)).reshape(NW, KD, CH)

    src_p0 = jnp.pad(src, (0, EPAD_P - e)).reshape(NS, KP, CH)
    src_p = jnp.stack([src_p0, src_p0 + N_NODES])
    dst_p = jnp.pad(dst, (0, EPAD_P - e)).reshape(NS, KP, CH)
    w_p = jnp.pad(w, (0, EPAD_P - e)).reshape(NS, KP, CH)

    degp = _sc_deg(dst_d, w_d)
    degp_t = degp.T
    yflat = _tc_y(x, W.astype(jnp.float32), degp_t)
    zp = _sc_propagate(src_p, dst_p, w_p, yflat)
    h, gemb = _tc_final(zp, yflat, degp_t,
                        b.reshape(1, D).astype(jnp.float32),
                        batch.reshape(1, N_NODES).astype(jnp.int32))
    return (h, gemb)
